# Initial kernel scaffold; baseline (speedup 1.0000x reference)
import jax
import jax.numpy as jnp
from jax.experimental import pallas as pl


def kernel(x, table, pos):
    raise NotImplementedError("write your pallas kernel here")


# speedup vs baseline: 1.7528x; 1.7528x over previous
import functools

import jax
import jax.numpy as jnp
from jax import lax
from jax.experimental import pallas as pl
from jax.experimental.pallas import tpu as pltpu
from jax.experimental.pallas import tpu_sc as plsc

VOCAB = 100000
EMBED = 128
WINDOW = 200
BATCH = 1024

N = BATCH * WINDOW
NC = 2
NS = 16
NW = NC * NS
ROWS_W = N // NW
CHUNK = 40
NCHUNK = ROWS_W // CHUNK
GROUPS = EMBED // 16

_mesh = plsc.VectorSubcoreMesh(core_axis_name="c", subcore_axis_name="s")


@functools.partial(
    pl.kernel,
    mesh=_mesh,
    out_type=jax.ShapeDtypeStruct((N, EMBED), jnp.float32),
    scratch_types=[
        pltpu.VMEM((NCHUNK, CHUNK), jnp.int32),
        pltpu.VMEM((WINDOW, EMBED), jnp.float32),
        pltpu.VMEM((CHUNK, EMBED), jnp.float32),
        pltpu.SemaphoreType.DMA,
    ],
)
def _embed(x_hbm, table_hbm, pos_hbm, out_hbm, idx_v, pos_v, buf, sem):
    wid = lax.axis_index("s") * NC + lax.axis_index("c")
    base = wid * ROWS_W

    pltpu.sync_copy(x_hbm.at[pl.ds(wid * NCHUNK, NCHUNK)], idx_v)
    pltpu.sync_copy(pos_hbm, pos_v)

    def chunk_body(j, carry):
        pltpu.async_copy(table_hbm.at[idx_v.at[j]], buf, sem).wait()
        poff = lax.rem(j, WINDOW // CHUNK) * CHUNK

        def row_body(i, c):
            for g in range(GROUPS):
                s = pl.ds(g * 16, 16)
                plsc.addupdate(buf.at[i, s], pos_v[poff + i, s])
            return c

        lax.fori_loop(0, CHUNK, row_body, 0)
        pltpu.sync_copy(buf, out_hbm.at[pl.ds(base + j * CHUNK, CHUNK)])
        return carry

    lax.fori_loop(0, NCHUNK, chunk_body, 0)


def kernel(x, table, pos):
    xr = x.astype(jnp.int32).reshape(NW * NCHUNK, CHUNK)
    out = _embed(xr, table, pos)
    return out.reshape(BATCH, WINDOW, EMBED)


# speedup vs baseline: 2.8216x; 1.6097x over previous
import functools

import jax
import jax.numpy as jnp
from jax import lax
from jax.experimental import pallas as pl
from jax.experimental.pallas import tpu as pltpu
from jax.experimental.pallas import tpu_sc as plsc

VOCAB = 100000
EMBED = 128
WINDOW = 200
BATCH = 1024

N = BATCH * WINDOW
NC = 2
NS = 16
NW = NC * NS
ROWS_W = N // NW
CHUNK = 40
NCHUNK = ROWS_W // CHUNK
GROUPS = EMBED // 16
NBUF = 4
PHASES = WINDOW // CHUNK

_mesh = plsc.VectorSubcoreMesh(core_axis_name="c", subcore_axis_name="s")


@functools.partial(
    pl.kernel,
    mesh=_mesh,
    out_type=jax.ShapeDtypeStruct((N, EMBED), jnp.float32),
    scratch_types=(
        [pltpu.VMEM((NCHUNK, CHUNK), jnp.int32),
         pltpu.VMEM((WINDOW, EMBED), jnp.float32)]
        + [pltpu.VMEM((CHUNK, EMBED), jnp.float32) for _ in range(NBUF)]
        + [pltpu.SemaphoreType.DMA for _ in range(2 * NBUF)]
    ),
)
def _embed(x_hbm, table_hbm, pos_hbm, out_hbm, idx_v, pos_v, *rest):
    bufs = rest[:NBUF]
    gsems = rest[NBUF:2 * NBUF]
    ssems = rest[2 * NBUF:]
    wid = lax.axis_index("s") * NC + lax.axis_index("c")
    base = wid * ROWS_W

    pltpu.sync_copy(x_hbm.at[pl.ds(wid * NCHUNK, NCHUNK)], idx_v)
    pltpu.sync_copy(pos_hbm, pos_v)

    def start_gather(c, b):
        pltpu.async_copy(table_hbm.at[idx_v.at[c]], bufs[b], gsems[b])

    def wait_gather(c, b):
        pltpu.make_async_copy(table_hbm.at[idx_v.at[c]], bufs[b], gsems[b]).wait()

    def start_store(c, b):
        pltpu.async_copy(bufs[b], out_hbm.at[pl.ds(base + c * CHUNK, CHUNK)],
                         ssems[b])

    def wait_store(c, b):
        pltpu.make_async_copy(bufs[b], out_hbm.at[pl.ds(base + c * CHUNK, CHUNK)],
                              ssems[b]).wait()

    def compute(c, b):
        poff = lax.rem(c, PHASES) * CHUNK

        def row_body(i, carry):
            for g in range(GROUPS):
                s = pl.ds(g * 16, 16)
                plsc.addupdate(bufs[b].at[i, s], pos_v[poff + i, s])
            return carry

        lax.fori_loop(0, CHUNK, row_body, 0, unroll=4)

    for c in range(NBUF - 1):
        start_gather(c, c)

    start_gather(NBUF - 1, NBUF - 1)
    wait_gather(0, 0)
    compute(0, 0)
    start_store(0, 0)

    def loop_body(j, carry):
        for k in range(NBUF):
            c = 1 + j * NBUF + k
            b = (1 + k) % NBUF
            bl = k % NBUF
            wait_store(c - 1, bl)
            start_gather(c + NBUF - 1, bl)
            wait_gather(c, b)
            compute(c, b)
            start_store(c, b)
        return carry

    lax.fori_loop(0, (NCHUNK - NBUF) // NBUF, loop_body, 0)

    for c in range(NCHUNK - NBUF + 1, NCHUNK):
        b = c % NBUF
        wait_gather(c, b)
        compute(c, b)
        start_store(c, b)

    for c in range(NCHUNK - NBUF, NCHUNK):
        wait_store(c, c % NBUF)


def kernel(x, table, pos):
    xr = x.astype(jnp.int32).reshape(NW * NCHUNK, CHUNK)
    out = _embed(xr, table, pos)
    return out.reshape(BATCH, WINDOW, EMBED)


# speedup vs baseline: 3.2753x; 1.1608x over previous
import functools

import jax
import jax.numpy as jnp
from jax import lax
from jax.experimental import pallas as pl
from jax.experimental.pallas import tpu as pltpu
from jax.experimental.pallas import tpu_sc as plsc

VOCAB = 100000
EMBED = 128
WINDOW = 200
BATCH = 1024

N = BATCH * WINDOW
NC = 2
NS = 16
NW = NC * NS
ROWS_W = N // NW
CHUNK = 128
NCHUNK = ROWS_W // CHUNK
GROUPS = EMBED // 16
NBUF = 2

_mesh = plsc.VectorSubcoreMesh(core_axis_name="c", subcore_axis_name="s")


@functools.partial(
    pl.kernel,
    mesh=_mesh,
    out_type=jax.ShapeDtypeStruct((N, EMBED), jnp.float32),
    scratch_types=(
        [pltpu.VMEM((ROWS_W,), jnp.int32),
         pltpu.VMEM((WINDOW, EMBED), jnp.float32)]
        + [pltpu.VMEM((CHUNK, EMBED), jnp.float32) for _ in range(NBUF)]
        + [pltpu.SemaphoreType.DMA for _ in range(2 * NBUF)]
    ),
)
def _embed(x_hbm, table_hbm, pos_hbm, out_hbm, idx_v, pos_v, *rest):
    bufs = rest[:NBUF]
    gsems = rest[NBUF:2 * NBUF]
    ssems = rest[2 * NBUF:]
    wid = lax.axis_index("s") * NC + lax.axis_index("c")
    base = wid * ROWS_W

    pltpu.sync_copy(x_hbm.at[pl.ds(base, ROWS_W)], idx_v)
    pltpu.sync_copy(pos_hbm, pos_v)

    def _idx(c):
        return idx_v.at[pl.ds(c * CHUNK, CHUNK)]

    def start_gather(c, b):
        pltpu.async_copy(table_hbm.at[_idx(c)], bufs[b], gsems[b])

    def wait_gather(c, b):
        pltpu.make_async_copy(table_hbm.at[_idx(c)], bufs[b], gsems[b]).wait()

    def start_store(c, b):
        pltpu.async_copy(bufs[b], out_hbm.at[pl.ds(base + c * CHUNK, CHUNK)],
                         ssems[b])

    def wait_store(c, b):
        pltpu.make_async_copy(bufs[b], out_hbm.at[pl.ds(base + c * CHUNK, CHUNK)],
                              ssems[b]).wait()

    def compute(c, b):
        cbase = lax.rem(c * CHUNK, WINDOW)

        def row_body(i, carry):
            w = cbase + i
            w = lax.select(w >= WINDOW, w - WINDOW, w)
            for g in range(GROUPS):
                s = pl.ds(g * 16, 16)
                plsc.addupdate(bufs[b].at[i, s], pos_v[w, s])
            return carry

        lax.fori_loop(0, CHUNK, row_body, 0, unroll=4)

    for c in range(NBUF - 1):
        start_gather(c, c)

    start_gather(NBUF - 1, NBUF - 1)
    wait_gather(0, 0)
    compute(0, 0)
    start_store(0, 0)

    def loop_body(j, carry):
        for k in range(NBUF):
            c = 1 + j * NBUF + k
            b = (1 + k) % NBUF
            bl = k % NBUF
            wait_store(c - 1, bl)
            start_gather(c + NBUF - 1, bl)
            wait_gather(c, b)
            compute(c, b)
            start_store(c, b)
        return carry

    lax.fori_loop(0, (NCHUNK - NBUF) // NBUF, loop_body, 0)

    for c in range(NCHUNK - NBUF + 1, NCHUNK):
        b = c % NBUF
        wait_gather(c, b)
        compute(c, b)
        start_store(c, b)

    for c in range(NCHUNK - NBUF, NCHUNK):
        wait_store(c, c % NBUF)


def kernel(x, table, pos):
    xr = x.astype(jnp.int32).reshape(N)
    out = _embed(xr, table, pos)
    return out.reshape(BATCH, WINDOW, EMBED)


# speedup vs baseline: 3.2807x; 1.0016x over previous
import functools

import jax
import jax.numpy as jnp
from jax import lax
from jax.experimental import pallas as pl
from jax.experimental.pallas import tpu as pltpu
from jax.experimental.pallas import tpu_sc as plsc

VOCAB = 100000
EMBED = 128
WINDOW = 200
BATCH = 1024

N = BATCH * WINDOW
NC = 2
NS = 16
NW = NC * NS
ROWS_W = N // NW
CHUNK = 128
NCHUNK = ROWS_W // CHUNK
GROUPS = EMBED // 16
NBUF = 4

_mesh = plsc.VectorSubcoreMesh(core_axis_name="c", subcore_axis_name="s")


@functools.partial(
    pl.kernel,
    mesh=_mesh,
    out_type=jax.ShapeDtypeStruct((N, EMBED), jnp.float32),
    scratch_types=(
        [pltpu.VMEM((ROWS_W,), jnp.int32),
         pltpu.VMEM((WINDOW, EMBED), jnp.float32)]
        + [pltpu.VMEM((CHUNK, EMBED), jnp.float32) for _ in range(NBUF)]
        + [pltpu.SemaphoreType.DMA for _ in range(2 * NBUF)]
    ),
)
def _embed(x_hbm, table_hbm, pos_hbm, out_hbm, idx_v, pos_v, *rest):
    bufs = rest[:NBUF]
    gsems = rest[NBUF:2 * NBUF]
    ssems = rest[2 * NBUF:]
    wid = lax.axis_index("s") * NC + lax.axis_index("c")
    base = wid * ROWS_W

    pltpu.sync_copy(x_hbm.at[pl.ds(base, ROWS_W)], idx_v)
    pltpu.sync_copy(pos_hbm, pos_v)

    def _idx(c):
        return idx_v.at[pl.ds(c * CHUNK, CHUNK)]

    def start_gather(c, b):
        pltpu.async_copy(table_hbm.at[_idx(c)], bufs[b], gsems[b])

    def wait_gather(c, b):
        pltpu.make_async_copy(table_hbm.at[_idx(c)], bufs[b], gsems[b]).wait()

    def start_store(c, b):
        pltpu.async_copy(bufs[b], out_hbm.at[pl.ds(base + c * CHUNK, CHUNK)],
                         ssems[b])

    def wait_store(c, b):
        pltpu.make_async_copy(bufs[b], out_hbm.at[pl.ds(base + c * CHUNK, CHUNK)],
                              ssems[b]).wait()

    def compute(c, b):
        cbase = lax.rem(c * CHUNK, WINDOW)

        def row_body(i, carry):
            w = cbase + i
            w = lax.select(w >= WINDOW, w - WINDOW, w)
            for g in range(GROUPS):
                s = pl.ds(g * 16, 16)
                plsc.addupdate(bufs[b].at[i, s], pos_v[w, s])
            return carry

        lax.fori_loop(0, CHUNK, row_body, 0, unroll=4)

    for c in range(NBUF - 1):
        start_gather(c, c)

    start_gather(NBUF - 1, NBUF - 1)
    wait_gather(0, 0)
    compute(0, 0)
    start_store(0, 0)

    def loop_body(j, carry):
        for k in range(NBUF):
            c = 1 + j * NBUF + k
            b = (1 + k) % NBUF
            bl = k % NBUF
            wait_store(c - 1, bl)
            start_gather(c + NBUF - 1, bl)
            wait_gather(c, b)
            compute(c, b)
            start_store(c, b)
        return carry

    steady_iters = (NCHUNK - NBUF) // NBUF
    lax.fori_loop(0, steady_iters, loop_body, 0)

    for c in range(steady_iters * NBUF + 1, NCHUNK):
        b = c % NBUF
        if c + NBUF - 1 < NCHUNK:
            wait_store(c - 1, (c - 1) % NBUF)
            start_gather(c + NBUF - 1, (c - 1) % NBUF)
        wait_gather(c, b)
        compute(c, b)
        start_store(c, b)

    for c in range(NCHUNK - NBUF, NCHUNK):
        wait_store(c, c % NBUF)


def kernel(x, table, pos):
    xr = x.astype(jnp.int32).reshape(N)
    out = _embed(xr, table, pos)
    return out.reshape(BATCH, WINDOW, EMBED)


# speedup vs baseline: 6.1415x; 1.8720x over previous
import functools

import jax
import jax.numpy as jnp
from jax import lax
from jax.experimental import pallas as pl
from jax.experimental.pallas import tpu as pltpu
from jax.experimental.pallas import tpu_sc as plsc

VOCAB = 100000
EMBED = 128
WINDOW = 200
BATCH = 1024

N = BATCH * WINDOW
NC = 2
NS = 16
NW = NC * NS
ROWS_W = N // NW
CHUNK = 128
NCHUNK = ROWS_W // CHUNK
GROUPS = EMBED // 16
NBUF = 3

_mesh = plsc.VectorSubcoreMesh(core_axis_name="c", subcore_axis_name="s")


@functools.partial(
    pl.kernel,
    mesh=_mesh,
    out_type=jax.ShapeDtypeStruct((N, EMBED), jnp.float32),
    scratch_types=(
        [pltpu.VMEM((ROWS_W,), jnp.int32),
         pltpu.VMEM((2 * WINDOW, EMBED), jnp.float32)]
        + [pltpu.VMEM((CHUNK, EMBED), jnp.float32) for _ in range(NBUF)]
        + [pltpu.SemaphoreType.DMA for _ in range(2 * NBUF)]
    ),
)
def _embed(x_hbm, table_hbm, pos_hbm, out_hbm, idx_v, pos_v, *rest):
    bufs = rest[:NBUF]
    gsems = rest[NBUF:2 * NBUF]
    ssems = rest[2 * NBUF:]
    wid = lax.axis_index("s") * NC + lax.axis_index("c")
    base = wid * ROWS_W

    pltpu.sync_copy(x_hbm.at[pl.ds(base, ROWS_W)], idx_v)
    pltpu.sync_copy(pos_hbm, pos_v.at[pl.ds(0, WINDOW)])
    pltpu.sync_copy(pos_hbm, pos_v.at[pl.ds(WINDOW, WINDOW)])

    def _idx(c):
        return idx_v.at[pl.ds(c * CHUNK, CHUNK)]

    def start_gather(c, b):
        pltpu.async_copy(table_hbm.at[_idx(c)], bufs[b], gsems[b])

    def wait_gather(c, b):
        pltpu.make_async_copy(table_hbm.at[_idx(c)], bufs[b], gsems[b]).wait()

    def start_store(c, b):
        pltpu.async_copy(bufs[b], out_hbm.at[pl.ds(base + c * CHUNK, CHUNK)],
                         ssems[b])

    def wait_store(c, b):
        pltpu.make_async_copy(bufs[b], out_hbm.at[pl.ds(base + c * CHUNK, CHUNK)],
                              ssems[b]).wait()

    def compute(c, b):
        cbase = lax.rem(c * CHUNK, WINDOW)
        buf = bufs[b]

        @plsc.parallel_loop(0, CHUNK, unroll=4)
        def row_body(i):
            w = cbase + i
            for g in range(GROUPS):
                s = pl.ds(g * 16, 16)
                plsc.addupdate(buf.at[i, s], pos_v[w, s])

    for c in range(NBUF - 1):
        start_gather(c, c)

    start_gather(NBUF - 1, NBUF - 1)
    wait_gather(0, 0)
    compute(0, 0)
    start_store(0, 0)

    def loop_body(j, carry):
        for k in range(NBUF):
            c = 1 + j * NBUF + k
            b = (1 + k) % NBUF
            bl = k % NBUF
            wait_store(c - 1, bl)
            start_gather(c + NBUF - 1, bl)
            wait_gather(c, b)
            compute(c, b)
            start_store(c, b)
        return carry

    steady_iters = (NCHUNK - NBUF) // NBUF
    lax.fori_loop(0, steady_iters, loop_body, 0)

    for c in range(steady_iters * NBUF + 1, NCHUNK):
        b = c % NBUF
        if c + NBUF - 1 < NCHUNK:
            wait_store(c - 1, (c - 1) % NBUF)
            start_gather(c + NBUF - 1, (c - 1) % NBUF)
        wait_gather(c, b)
        compute(c, b)
        start_store(c, b)

    for c in range(NCHUNK - NBUF, NCHUNK):
        wait_store(c, c % NBUF)


def kernel(x, table, pos):
    xr = x.astype(jnp.int32).reshape(N)
    out = _embed(xr, table, pos)
    return out.reshape(BATCH, WINDOW, EMBED)


# speedup vs baseline: 7.2679x; 1.1834x over previous
import functools

import jax
import jax.numpy as jnp
from jax import lax
from jax.experimental import pallas as pl
from jax.experimental.pallas import tpu as pltpu
from jax.experimental.pallas import tpu_sc as plsc

VOCAB = 100000
EMBED = 128
WINDOW = 200
BATCH = 1024

N = BATCH * WINDOW
NC = 2
NS = 16
NW = NC * NS
ROWS_W = N // NW
CHUNK = 128
NCHUNK = ROWS_W // CHUNK
GROUPS = EMBED // 16
NBUF = 4

_mesh = plsc.VectorSubcoreMesh(core_axis_name="c", subcore_axis_name="s")


@functools.partial(
    pl.kernel,
    mesh=_mesh,
    out_type=jax.ShapeDtypeStruct((N, EMBED), jnp.float32),
    scratch_types=(
        [pltpu.VMEM((ROWS_W,), jnp.int32),
         pltpu.VMEM((2 * WINDOW, EMBED), jnp.float32)]
        + [pltpu.VMEM((CHUNK, EMBED), jnp.float32) for _ in range(NBUF)]
        + [pltpu.SemaphoreType.DMA for _ in range(2 * NBUF + 1)]
    ),
)
def _embed(x_hbm, table_hbm, pos_hbm, out_hbm, idx_v, pos_v, *rest):
    bufs = rest[:NBUF]
    gsems = rest[NBUF:2 * NBUF]
    ssems = rest[2 * NBUF:3 * NBUF]
    psem = rest[3 * NBUF]
    wid = lax.axis_index("s") * NC + lax.axis_index("c")
    base = wid * ROWS_W

    pos_lo = pltpu.async_copy(pos_hbm, pos_v.at[pl.ds(0, WINDOW)], psem)
    pos_hi = pltpu.async_copy(pos_hbm, pos_v.at[pl.ds(WINDOW, WINDOW)], psem)
    pltpu.sync_copy(x_hbm.at[pl.ds(base, ROWS_W)], idx_v)

    def _idx(c):
        return idx_v.at[pl.ds(c * CHUNK, CHUNK)]

    def start_gather(c, b):
        pltpu.async_copy(table_hbm.at[_idx(c)], bufs[b], gsems[b])

    def wait_gather(c, b):
        pltpu.make_async_copy(table_hbm.at[_idx(c)], bufs[b], gsems[b]).wait()

    def start_store(c, b):
        pltpu.async_copy(bufs[b], out_hbm.at[pl.ds(base + c * CHUNK, CHUNK)],
                         ssems[b])

    def wait_store(c, b):
        pltpu.make_async_copy(bufs[b], out_hbm.at[pl.ds(base + c * CHUNK, CHUNK)],
                              ssems[b]).wait()

    def compute(c, b):
        cbase = lax.rem(c * CHUNK, WINDOW)
        buf = bufs[b]

        @plsc.parallel_loop(0, CHUNK, unroll=4)
        def row_body(i):
            w = cbase + i
            for g in range(GROUPS):
                s = pl.ds(g * 16, 16)
                plsc.addupdate(buf.at[i, s], pos_v[w, s])

    for c in range(NBUF - 1):
        start_gather(c, c)

    start_gather(NBUF - 1, NBUF - 1)
    wait_gather(0, 0)
    pos_lo.wait()
    pos_hi.wait()
    compute(0, 0)
    start_store(0, 0)

    def loop_body(j, carry):
        for k in range(NBUF):
            c = 1 + j * NBUF + k
            b = (1 + k) % NBUF
            bl = k % NBUF
            wait_gather(c, b)
            compute(c, b)
            start_store(c, b)
            wait_store(c - 1, bl)
            start_gather(c + NBUF - 1, bl)
        return carry

    steady_iters = (NCHUNK - NBUF) // NBUF
    lax.fori_loop(0, steady_iters, loop_body, 0)

    for c in range(steady_iters * NBUF + 1, NCHUNK):
        b = c % NBUF
        wait_gather(c, b)
        compute(c, b)
        start_store(c, b)
        if c + NBUF - 1 < NCHUNK:
            wait_store(c - 1, (c - 1) % NBUF)
            start_gather(c + NBUF - 1, (c - 1) % NBUF)

    for c in range(NCHUNK - NBUF, NCHUNK):
        wait_store(c, c % NBUF)


def kernel(x, table, pos):
    xr = x.astype(jnp.int32).reshape(N)
    out = _embed(xr, table, pos)
    return out.reshape(BATCH, WINDOW, EMBED)
